# baseline (speedup 1.0000x reference)
import jax
import jax.numpy as jnp
from jax.experimental import pallas as pl

N = 10000
E = 320000
G = 64
D_IN = 128
H = 256

_EB = 4000


def _edge_mlp_body(ea_ref, w1_ref, b1_ref, w2_ref, b2_ref, l1w_ref, l1b_ref,
                   l2w_ref, l2b_ref, e1_ref, e2_ref):
    t = jnp.dot(ea_ref[...], w1_ref[...], preferred_element_type=jnp.float32)
    t = jnp.maximum(t + b1_ref[...], 0.0)
    ea = jnp.dot(t, w2_ref[...], preferred_element_type=jnp.float32) + b2_ref[...]
    e1_ref[...] = jnp.dot(ea, l1w_ref[...], preferred_element_type=jnp.float32) + l1b_ref[...]
    e2_ref[...] = jnp.dot(ea, l2w_ref[...], preferred_element_type=jnp.float32) + l2b_ref[...]


def _edge_features(edge_attr, em_w1, em_b1, em_w2, em_b2, c1_lw, c1_lb,
                   c2_lw, c2_lb):
    nblk = E // _EB
    return pl.pallas_call(
        _edge_mlp_body,
        grid=(nblk,),
        in_specs=[
            pl.BlockSpec((_EB, 16), lambda i: (i, 0)),
            pl.BlockSpec((16, H), lambda i: (0, 0)),
            pl.BlockSpec((1, H), lambda i: (0, 0)),
            pl.BlockSpec((H, H), lambda i: (0, 0)),
            pl.BlockSpec((1, H), lambda i: (0, 0)),
            pl.BlockSpec((H, D_IN), lambda i: (0, 0)),
            pl.BlockSpec((1, D_IN), lambda i: (0, 0)),
            pl.BlockSpec((H, H), lambda i: (0, 0)),
            pl.BlockSpec((1, H), lambda i: (0, 0)),
        ],
        out_specs=[
            pl.BlockSpec((_EB, D_IN), lambda i: (i, 0)),
            pl.BlockSpec((_EB, H), lambda i: (i, 0)),
        ],
        out_shape=[
            jax.ShapeDtypeStruct((E, D_IN), jnp.float32),
            jax.ShapeDtypeStruct((E, H), jnp.float32),
        ],
    )(edge_attr, em_w1, em_b1, em_w2, em_b2, c1_lw, c1_lb, c2_lw, c2_lb)


_NB = 2000


def _node_mlp_body(x_ref, agg_ref, w1_ref, b1_ref, w2_ref, b2_ref, out_ref):
    h = x_ref[...] + agg_ref[...]
    u = jnp.maximum(
        jnp.dot(h, w1_ref[...], preferred_element_type=jnp.float32) + b1_ref[...], 0.0)
    v = jnp.dot(u, w2_ref[...], preferred_element_type=jnp.float32) + b2_ref[...]
    out_ref[...] = jnp.maximum(v, 0.0)


def _node_mlp(x, agg, w1, b1, w2, b2):
    nblk = N // _NB
    din = x.shape[1]
    return pl.pallas_call(
        _node_mlp_body,
        grid=(nblk,),
        in_specs=[
            pl.BlockSpec((_NB, din), lambda i: (i, 0)),
            pl.BlockSpec((_NB, din), lambda i: (i, 0)),
            pl.BlockSpec((din, H), lambda i: (0, 0)),
            pl.BlockSpec((1, H), lambda i: (0, 0)),
            pl.BlockSpec((H, H), lambda i: (0, 0)),
            pl.BlockSpec((1, H), lambda i: (0, 0)),
        ],
        out_specs=pl.BlockSpec((_NB, H), lambda i: (i, 0)),
        out_shape=jax.ShapeDtypeStruct((N, H), jnp.float32),
    )(x, agg, w1, b1, w2, b2)


def _head_body(sums_ref, cnts_ref, l1w_ref, l1b_ref, l2w_ref, l2b_ref,
               hw_ref, hb_ref, out_ref):
    g = sums_ref[...] / jnp.maximum(cnts_ref[...], 1.0)
    g1 = jnp.maximum(
        jnp.dot(g, l1w_ref[...], preferred_element_type=jnp.float32) + l1b_ref[...], 0.0)
    g2 = jnp.maximum(
        jnp.dot(g1, l2w_ref[...], preferred_element_type=jnp.float32) + l2b_ref[...], 0.0)
    out_ref[...] = jnp.dot(g2, hw_ref[...], preferred_element_type=jnp.float32) + hb_ref[...]


def _head(sums, cnts, l1w, l1b, l2w, l2b, hw, hb):
    return pl.pallas_call(
        _head_body,
        in_specs=[
            pl.BlockSpec((G, H), lambda: (0, 0)),
            pl.BlockSpec((G, 1), lambda: (0, 0)),
            pl.BlockSpec((H, 128), lambda: (0, 0)),
            pl.BlockSpec((1, 128), lambda: (0, 0)),
            pl.BlockSpec((128, 64), lambda: (0, 0)),
            pl.BlockSpec((1, 64), lambda: (0, 0)),
            pl.BlockSpec((64, 3), lambda: (0, 0)),
            pl.BlockSpec((1, 3), lambda: (0, 0)),
        ],
        out_specs=pl.BlockSpec((G, 3), lambda: (0, 0)),
        out_shape=jax.ShapeDtypeStruct((G, 3), jnp.float32),
    )(sums, cnts, l1w, l1b, l2w, l2b, hw, hb)


def kernel(x, edge_index, edge_attr, batch,
           em_w1, em_b1, em_w2, em_b2,
           c1_lw, c1_lb, c1_w1, c1_b1, c1_w2, c1_b2,
           c2_lw, c2_lb, c2_w1, c2_b1, c2_w2, c2_b2,
           l1_w, l1_b, l2_w, l2_b, hS_w, hS_b, hP_w, hP_b, hN_w, hN_b):
    src = edge_index[0]
    dst = edge_index[1]

    e1, e2 = _edge_features(edge_attr, em_w1, em_b1[None], em_w2, em_b2[None],
                            c1_lw, c1_lb[None], c2_lw, c2_lb[None])

    m1 = jax.nn.relu(x[src] + e1)
    agg1 = jnp.zeros_like(x).at[dst].add(m1)
    h1 = _node_mlp(x, agg1, c1_w1, c1_b1[None], c1_w2, c1_b2[None])

    m2 = jax.nn.relu(h1[src] + e2)
    agg2 = jnp.zeros_like(h1).at[dst].add(m2)
    h2 = _node_mlp(h1, agg2, c2_w1, c2_b1[None], c2_w2, c2_b2[None])

    sums = jax.ops.segment_sum(h2, batch, num_segments=G)
    cnts = jax.ops.segment_sum(jnp.ones((N,), jnp.float32), batch,
                               num_segments=G)

    hw = jnp.concatenate([hS_w, hP_w, hN_w], axis=1)
    hb = jnp.concatenate([hS_b, hP_b, hN_b]).reshape(1, 3)
    out = _head(sums, cnts[:, None], l1_w, l1_b[None], l2_w, l2_b[None], hw, hb)
    return out[:, 0], out[:, 1], out[:, 2]
